# baseline (speedup 1.0000x reference)
import jax
import jax.numpy as jnp
from jax import lax
from jax.experimental import pallas as pl
from jax.experimental.pallas import tpu as pltpu
from jax.experimental.pallas import tpu_sc as plsc

N_NODES = 10000
N_EDGES = 320000
D_FEAT = 128

_CHUNK = 80
_NCHUNK = N_EDGES // _CHUNK
_NW = 32
_NG = _NCHUNK // _NW


def _gather_incident_kernel(table_hbm, icat_hbm, out_hbm,
                            tbl_sh, icat, rows_cat, semi, semg_d, semg_s, semo):
    sid = lax.axis_index("s")
    wid = sid * 2 + lax.axis_index("c")

    rows_per_tile = 624
    pltpu.sync_copy(table_hbm.at[pl.ds(sid * rows_per_tile, rows_per_tile)],
                    tbl_sh.at[pl.ds(sid * rows_per_tile, rows_per_tile)])

    @pl.when(sid == 0)
    def _():
        pltpu.sync_copy(table_hbm.at[pl.ds(16 * rows_per_tile, N_NODES - 16 * rows_per_tile)],
                        tbl_sh.at[pl.ds(16 * rows_per_tile, N_NODES - 16 * rows_per_tile)])

    plsc.subcore_barrier()

    def chunk_id(g):
        return g * _NW + wid

    def stage_a(g, b):
        @pl.when(jnp.logical_and(g >= 0, g < _NG))
        def _():
            pltpu.async_copy(icat_hbm.at[chunk_id(g)], icat.at[b], semi[b])

    def stage_b(g, b):
        @pl.when(jnp.logical_and(g >= 0, g < _NG))
        def _():
            pltpu.make_async_copy(icat_hbm.at[0], icat.at[b], semi[b]).wait()

            @pl.when(g >= 2)
            def _():
                pltpu.make_async_copy(rows_cat.at[b], out_hbm.at[pl.ds(0, 2 * _CHUNK)], semo[b]).wait()

            pltpu.async_copy(tbl_sh.at[icat.at[b, 0]], rows_cat.at[b, pl.ds(0, _CHUNK)], semg_d[b])
            pltpu.async_copy(tbl_sh.at[icat.at[b, 1]], rows_cat.at[b, pl.ds(_CHUNK, _CHUNK)], semg_s[b])

    def stage_c(g, b):
        @pl.when(jnp.logical_and(g >= 0, g < _NG))
        def _():
            pltpu.make_async_copy(tbl_sh.at[icat.at[b, 0]], rows_cat.at[b, pl.ds(0, _CHUNK)], semg_d[b]).wait()
            pltpu.make_async_copy(tbl_sh.at[icat.at[b, 1]], rows_cat.at[b, pl.ds(_CHUNK, _CHUNK)], semg_s[b]).wait()
            pltpu.async_copy(rows_cat.at[b], out_hbm.at[pl.ds(2 * _CHUNK * chunk_id(g), 2 * _CHUNK)], semo[b])

    def step(s, carry):
        for p in range(2):
            g = s * 2 + p
            stage_b(g - 1, (p + 1) % 2)
            stage_c(g - 2, p % 2)
            stage_a(g, p % 2)
        return carry

    lax.fori_loop(0, (_NG + 2 + 1) // 2, step, 0)

    for b in range(2):
        pltpu.make_async_copy(rows_cat.at[b], out_hbm.at[pl.ds(0, 2 * _CHUNK)], semo[b]).wait()


@jax.jit
def kernel(node_feature, edge_src, edge_dst):
    icat_all = jnp.stack([edge_dst, edge_src], axis=1).reshape(_NCHUNK, 2, _CHUNK)

    mesh = plsc.VectorSubcoreMesh(core_axis_name="c", subcore_axis_name="s")
    run = pl.kernel(
        _gather_incident_kernel,
        out_type=jax.ShapeDtypeStruct((2 * N_EDGES, D_FEAT), jnp.float32),
        mesh=mesh,
        scratch_types=[
            pltpu.VMEM_SHARED((N_NODES, D_FEAT), jnp.float32),
            pltpu.VMEM((2, 2, _CHUNK), jnp.int32),
            pltpu.VMEM((2, 2 * _CHUNK, D_FEAT), jnp.float32),
            [pltpu.SemaphoreType.DMA] * 2,
            [pltpu.SemaphoreType.DMA] * 2,
            [pltpu.SemaphoreType.DMA] * 2,
            [pltpu.SemaphoreType.DMA] * 2,
        ],
    )
    interleaved = run(node_feature, icat_all)
    return interleaved.reshape(N_EDGES, 2 * D_FEAT)
